# baseline (speedup 1.0000x reference)
import functools

import jax
import jax.numpy as jnp
from jax import lax
from jax.experimental import pallas as pl
from jax.experimental.pallas import tpu as pltpu
from jax.experimental.pallas import tpu_sc as plsc

BATCH = 16384
NROWS = 1000001
DIM = 32


@functools.lru_cache(maxsize=None)
def _build_gather(batch, dim, nrows):
    info = plsc.get_sparse_core_info()
    nw = info.num_cores * info.num_subcores
    bpw = batch // nw
    mesh = plsc.VectorSubcoreMesh(core_axis_name="c", subcore_axis_name="s")

    @functools.partial(
        pl.kernel,
        mesh=mesh,
        out_type=jax.ShapeDtypeStruct((dim, batch), jnp.float32),
        scratch_types=[
            pltpu.VMEM((bpw,), jnp.int32),
            pltpu.VMEM((dim, bpw), jnp.float32),
            pltpu.SemaphoreType.DMA,
        ],
        compiler_params=pltpu.CompilerParams(use_tc_tiling_on_sc=False),
    )
    def gather(idx_hbm, wt_hbm, out_hbm, idx_v, block_v, sem):
        wid = lax.axis_index("s") * info.num_cores + lax.axis_index("c")
        base = wid * bpw
        pltpu.sync_copy(idx_hbm.at[pl.ds(base, bpw)], idx_v)

        def body(j, _):
            pltpu.async_copy(wt_hbm.at[j].at[idx_v], block_v.at[j], sem)
            return _

        lax.fori_loop(0, dim, body, 0)

        def drain(j, _):
            pltpu.make_async_copy(
                wt_hbm.at[j].at[idx_v], block_v.at[j], sem
            ).wait()
            return _

        lax.fori_loop(0, dim, drain, 0)
        pltpu.sync_copy(block_v, out_hbm.at[:, pl.ds(base, bpw)])

    return gather


def kernel(indices, emb_weight):
    out_t = _build_gather(BATCH, DIM, NROWS)(
        indices.astype(jnp.int32), emb_weight.T
    )
    return out_t.T


# speedup vs baseline: 19.3335x; 19.3335x over previous
import functools

import jax
import jax.numpy as jnp
from jax import lax
from jax.experimental import pallas as pl
from jax.experimental.pallas import tpu as pltpu
from jax.experimental.pallas import tpu_sc as plsc

BATCH = 16384
NROWS = 1000001
DIM = 32
ROWPAD = 1000064

NCOLS = 7813
MAIN_COLS = 7808
CHUNK = 4
LAST_W = NROWS - (NCOLS - 1) * 128


@functools.lru_cache(maxsize=None)
def _build():
    info = plsc.get_sparse_core_info()
    nw = info.num_cores * info.num_subcores
    bpw = BATCH // nw
    cpw = MAIN_COLS // nw
    nchunks = cpw // CHUNK
    mesh = plsc.VectorSubcoreMesh(core_axis_name="c", subcore_axis_name="s")

    @functools.partial(
        pl.kernel,
        mesh=mesh,
        out_type=jax.ShapeDtypeStruct((DIM * ROWPAD,), jnp.float32),
        scratch_types=[
            pltpu.VMEM((2, DIM, CHUNK * 128), jnp.float32),
            pltpu.SemaphoreType.DMA,
            pltpu.SemaphoreType.DMA,
        ],
    )
    def _detile(wt_hbm, wtail_hbm, lin_hbm, buf, sem_r, sem_w):
        wid = lax.axis_index("s") * info.num_cores + lax.axis_index("c")
        col0 = wid * cpw

        def chunk_src(k):
            return wt_hbm.at[:, pl.ds((col0 + k * CHUNK) * 128, CHUNK * 128)]

        pltpu.async_copy(chunk_src(0), buf.at[0], sem_r)
        pltpu.async_copy(chunk_src(1), buf.at[1], sem_r)

        def body(k, carry):
            s = lax.rem(k, 2)
            pltpu.make_async_copy(chunk_src(0), buf.at[s], sem_r).wait()
            base = (col0 + k * CHUNK) * 128
            for j in range(DIM):
                pltpu.async_copy(
                    buf.at[s, j],
                    lin_hbm.at[pl.ds(j * ROWPAD + base, CHUNK * 128)],
                    sem_w,
                )

            @pl.when(k + 2 < nchunks)
            def _refill():
                pltpu.make_async_copy(chunk_src(0), buf.at[s], sem_w).wait()
                pltpu.async_copy(chunk_src(k + 2), buf.at[s], sem_r)

            return carry

        lax.fori_loop(0, nchunks, body, 0)
        pltpu.make_async_copy(chunk_src(0), buf.at[0], sem_w).wait()
        pltpu.make_async_copy(chunk_src(0), buf.at[1], sem_w).wait()

        @pl.when(wid < 4)
        def _():
            t = MAIN_COLS + wid
            pltpu.sync_copy(
                wt_hbm.at[:, pl.ds(t * 128, 128)], buf.at[0, :, pl.ds(0, 128)]
            )
            for j in range(DIM):
                pltpu.async_copy(
                    buf.at[0, j, pl.ds(0, 128)],
                    lin_hbm.at[pl.ds(j * ROWPAD + t * 128, 128)],
                    sem_w,
                )
            pltpu.make_async_copy(
                wt_hbm.at[:, pl.ds(0, 128)], buf.at[0, :, pl.ds(0, 128)], sem_w
            ).wait()

        @pl.when(wid == 4)
        def _():
            t = NCOLS - 1
            pltpu.sync_copy(wtail_hbm, buf.at[0, :, pl.ds(0, 128)])
            for j in range(DIM):
                pltpu.async_copy(
                    buf.at[0, j, pl.ds(0, 128)],
                    lin_hbm.at[pl.ds(j * ROWPAD + t * 128, 128)],
                    sem_w,
                )
            pltpu.make_async_copy(
                wt_hbm.at[:, pl.ds(0, 128)], buf.at[0, :, pl.ds(0, 128)], sem_w
            ).wait()

    @functools.partial(
        pl.kernel,
        mesh=mesh,
        out_type=jax.ShapeDtypeStruct((DIM, BATCH), jnp.float32),
        scratch_types=[
            pltpu.VMEM((bpw,), jnp.int32),
            pltpu.VMEM((DIM, bpw), jnp.int32),
            pltpu.VMEM((DIM, bpw), jnp.float32),
            pltpu.SemaphoreType.DMA,
        ],
        compiler_params=pltpu.CompilerParams(use_tc_tiling_on_sc=False),
    )
    def _gather(idx_hbm, lin_hbm, out_hbm, idx_v, off_v, block_v, sem):
        wid = lax.axis_index("s") * info.num_cores + lax.axis_index("c")
        base = wid * bpw
        pltpu.sync_copy(idx_hbm.at[pl.ds(base, bpw)], idx_v)

        def body(j, _):
            for v in range(bpw // 16):
                sl = pl.ds(v * 16, 16)
                off_v[j, sl] = idx_v[sl] + j * ROWPAD
            pltpu.async_copy(lin_hbm.at[off_v.at[j]], block_v.at[j], sem)
            return _

        lax.fori_loop(0, DIM, body, 0)

        def drain(j, _):
            pltpu.make_async_copy(
                lin_hbm.at[off_v.at[j]], block_v.at[j], sem
            ).wait()
            return _

        lax.fori_loop(0, DIM, drain, 0)
        pltpu.sync_copy(block_v, out_hbm.at[:, pl.ds(base, bpw)])

    return _detile, _gather


def kernel(indices, emb_weight):
    detile, gather = _build()
    wtail = jnp.pad(emb_weight[(NCOLS - 1) * 128 :].T, ((0, 0), (0, 128 - LAST_W)))
    lin = detile(emb_weight.T, wtail)
    out_t = gather(indices.astype(jnp.int32), lin)
    return out_t.T
